# baseline (speedup 1.0000x reference)
import functools

import jax
import jax.numpy as jnp
from jax import lax
from jax.experimental import pallas as pl
from jax.experimental.pallas import tpu as pltpu
from jax.experimental.pallas import tpu_sc as plsc

NC = 2
NS = 16
L = 16
NW = NC * NS

N = 100000
D = 128

TC_BLOCK = 2048
TC_GRID = (N + TC_BLOCK - 1) // TC_BLOCK
SIM_ROWS = 800
WROWS = SIM_ROWS // NW

_NEG_INF = float("-inf")
_IMAX = jnp.iinfo(jnp.int32).max


def _tc_sims(q_ref, keys_ref, out_ref):
    q = q_ref[0, :]
    qn = jnp.maximum(jnp.sqrt(jnp.sum(q * q)), 1e-8)
    k = keys_ref[...]
    dots = jnp.sum(k * q[None, :], axis=1)
    kn = jnp.maximum(jnp.sqrt(jnp.sum(k * k, axis=1)), 1e-8)
    sim = dots / (qn * kn)
    out_ref[...] = sim.reshape(TC_BLOCK // 128, 128)


def _sc_body(sims_hbm, values_hbm,
             rows_out, vals_out, idx_out,
             svmem, cand_v, cand_i, merged_v, merged_i, row_vmem,
             shared_v, shared_i):
    cid = lax.axis_index("c")
    sid = lax.axis_index("s")
    w = cid * NS + sid

    pltpu.sync_copy(sims_hbm.at[pl.ds(w * WROWS, WROWS)], svmem)

    wbase = w * (WROWS * 128) + lax.iota(jnp.int32, L)
    bv = jnp.full((L,), _NEG_INF, jnp.float32)
    bi = jnp.zeros((L,), jnp.int32)
    for r in range(WROWS):
        for c in range(128 // L):
            v = svmem[r, pl.ds(c * L, L)]
            gidx = wbase + (r * 128 + c * L)
            upd = (v > bv) & (gidx < N)
            bv = jnp.where(upd, v, bv)
            bi = jnp.where(upd, gidx, bi)

    cand_v[...] = bv
    cand_i[...] = bi
    pltpu.sync_copy(cand_v, shared_v.at[sid])
    pltpu.sync_copy(cand_i, shared_i.at[sid])
    plsc.subcore_barrier()

    @pl.when(sid == 0)
    def _():
        pltpu.sync_copy(shared_v, merged_v)
        pltpu.sync_copy(shared_i, merged_i)
        bv = merged_v[0, :]
        bi = merged_i[0, :]
        for t in range(1, NS):
            v = merged_v[t, :]
            i = merged_i[t, :]
            upd = (v > bv) | ((v == bv) & (i < bi))
            bv = jnp.where(upd, v, bv)
            bi = jnp.where(upd, i, bi)
        m = jnp.max(bv)
        midx = jnp.min(jnp.where(bv == m, bi, _IMAX))
        pltpu.sync_copy(values_hbm.at[pl.ds(midx, 1)], row_vmem)
        pltpu.sync_copy(row_vmem, rows_out.at[pl.ds(cid, 1)])
        cand_v[...] = jnp.full((L,), m, jnp.float32)
        cand_i[...] = jnp.full((L,), midx, jnp.int32)
        pltpu.sync_copy(cand_v, vals_out.at[cid])
        pltpu.sync_copy(cand_i, idx_out.at[cid])


@jax.jit
def kernel(query, keys, values):
    sims = pl.pallas_call(
        _tc_sims,
        grid=(TC_GRID,),
        in_specs=[
            pl.BlockSpec((1, D), lambda i: (0, 0)),
            pl.BlockSpec((TC_BLOCK, D), lambda i: (i, 0)),
        ],
        out_specs=pl.BlockSpec((TC_BLOCK // 128, 128), lambda i: (i, 0)),
        out_shape=jax.ShapeDtypeStruct((SIM_ROWS, 128), jnp.float32),
    )(query.reshape(1, D), keys)

    mesh = plsc.VectorSubcoreMesh(core_axis_name="c", subcore_axis_name="s")
    rows, vals, idxs = pl.kernel(
        _sc_body,
        out_type=(
            jax.ShapeDtypeStruct((NC, D), jnp.float32),
            jax.ShapeDtypeStruct((NC, L), jnp.float32),
            jax.ShapeDtypeStruct((NC, L), jnp.int32),
        ),
        mesh=mesh,
        compiler_params=pltpu.CompilerParams(
            use_tc_tiling_on_sc=False, needs_layout_passes=False),
        scratch_types=[
            pltpu.VMEM((WROWS, 128), jnp.float32),
            pltpu.VMEM((L,), jnp.float32),
            pltpu.VMEM((L,), jnp.int32),
            pltpu.VMEM((NS, L), jnp.float32),
            pltpu.VMEM((NS, L), jnp.int32),
            pltpu.VMEM((1, D), jnp.float32),
            pltpu.VMEM_SHARED((NS, L), jnp.float32),
            pltpu.VMEM_SHARED((NS, L), jnp.int32),
        ],
    )(sims, values)

    v0, v1 = vals[0, 0], vals[1, 0]
    i0, i1 = idxs[0, 0], idxs[1, 0]
    pick0 = (v0 > v1) | ((v0 == v1) & (i0 <= i1))
    return jnp.where(pick0, rows[0], rows[1])


# speedup vs baseline: 1.0332x; 1.0332x over previous
import functools

import jax
import jax.numpy as jnp
from jax import lax
from jax.experimental import pallas as pl
from jax.experimental.pallas import tpu as pltpu
from jax.experimental.pallas import tpu_sc as plsc

NC = 2
NS = 16
L = 16
NW = NC * NS

N = 100000
D = 128

TC_BLOCK = 2048
TC_GRID = (N + TC_BLOCK - 1) // TC_BLOCK
SIM_ROWS = 800
WROWS = SIM_ROWS // NW

_NEG_INF = float("-inf")
_IMAX = jnp.iinfo(jnp.int32).max


def _tc_sims(q_ref, keys_ref, out_ref):
    q = q_ref[0, :]
    qn = jnp.maximum(jnp.sqrt(jnp.sum(q * q)), 1e-8)
    k = keys_ref[...]
    q2 = q_ref[...].reshape(D, 1)
    dots = jnp.dot(k, q2, preferred_element_type=jnp.float32)
    ssq = jnp.dot(k * k, jnp.ones((D, 1), jnp.float32),
                  preferred_element_type=jnp.float32)
    kn = jnp.maximum(jnp.sqrt(ssq), 1e-8)
    sim = dots / (qn * kn)
    out_ref[...] = sim.reshape(TC_BLOCK // 128, 128)


def _sc_body(sims_hbm, values_hbm,
             rows_out, vals_out, idx_out,
             svmem, cand_v, cand_i, merged_v, merged_i, row_vmem,
             shared_v, shared_i):
    cid = lax.axis_index("c")
    sid = lax.axis_index("s")
    w = cid * NS + sid

    pltpu.sync_copy(sims_hbm.at[pl.ds(w * WROWS, WROWS)], svmem)

    wbase = w * (WROWS * 128) + lax.iota(jnp.int32, L)
    bv = jnp.full((L,), _NEG_INF, jnp.float32)
    bi = jnp.zeros((L,), jnp.int32)
    for r in range(WROWS):
        for c in range(128 // L):
            v = svmem[r, pl.ds(c * L, L)]
            gidx = wbase + (r * 128 + c * L)
            upd = (v > bv) & (gidx < N)
            bv = jnp.where(upd, v, bv)
            bi = jnp.where(upd, gidx, bi)

    cand_v[...] = bv
    cand_i[...] = bi
    pltpu.sync_copy(cand_v, shared_v.at[sid])
    pltpu.sync_copy(cand_i, shared_i.at[sid])
    plsc.subcore_barrier()

    @pl.when(sid == 0)
    def _():
        pltpu.sync_copy(shared_v, merged_v)
        pltpu.sync_copy(shared_i, merged_i)
        bv = merged_v[0, :]
        bi = merged_i[0, :]
        for t in range(1, NS):
            v = merged_v[t, :]
            i = merged_i[t, :]
            upd = (v > bv) | ((v == bv) & (i < bi))
            bv = jnp.where(upd, v, bv)
            bi = jnp.where(upd, i, bi)
        m = jnp.max(bv)
        midx = jnp.min(jnp.where(bv == m, bi, _IMAX))
        pltpu.sync_copy(values_hbm.at[pl.ds(midx, 1)], row_vmem)
        pltpu.sync_copy(row_vmem, rows_out.at[pl.ds(cid, 1)])
        cand_v[...] = jnp.full((L,), m, jnp.float32)
        cand_i[...] = jnp.full((L,), midx, jnp.int32)
        pltpu.sync_copy(cand_v, vals_out.at[cid])
        pltpu.sync_copy(cand_i, idx_out.at[cid])


@jax.jit
def kernel(query, keys, values):
    sims = pl.pallas_call(
        _tc_sims,
        grid=(TC_GRID,),
        in_specs=[
            pl.BlockSpec((1, D), lambda i: (0, 0)),
            pl.BlockSpec((TC_BLOCK, D), lambda i: (i, 0)),
        ],
        out_specs=pl.BlockSpec((TC_BLOCK // 128, 128), lambda i: (i, 0)),
        out_shape=jax.ShapeDtypeStruct((SIM_ROWS, 128), jnp.float32),
    )(query.reshape(1, D), keys)

    mesh = plsc.VectorSubcoreMesh(core_axis_name="c", subcore_axis_name="s")
    rows, vals, idxs = pl.kernel(
        _sc_body,
        out_type=(
            jax.ShapeDtypeStruct((NC, D), jnp.float32),
            jax.ShapeDtypeStruct((NC, L), jnp.float32),
            jax.ShapeDtypeStruct((NC, L), jnp.int32),
        ),
        mesh=mesh,
        compiler_params=pltpu.CompilerParams(
            use_tc_tiling_on_sc=False, needs_layout_passes=False),
        scratch_types=[
            pltpu.VMEM((WROWS, 128), jnp.float32),
            pltpu.VMEM((L,), jnp.float32),
            pltpu.VMEM((L,), jnp.int32),
            pltpu.VMEM((NS, L), jnp.float32),
            pltpu.VMEM((NS, L), jnp.int32),
            pltpu.VMEM((1, D), jnp.float32),
            pltpu.VMEM_SHARED((NS, L), jnp.float32),
            pltpu.VMEM_SHARED((NS, L), jnp.int32),
        ],
    )(sims, values)

    v0, v1 = vals[0, 0], vals[1, 0]
    i0, i1 = idxs[0, 0], idxs[1, 0]
    pick0 = (v0 > v1) | ((v0 == v1) & (i0 <= i1))
    return jnp.where(pick0, rows[0], rows[1])


# speedup vs baseline: 1.3296x; 1.2868x over previous
import functools

import jax
import jax.numpy as jnp
from jax import lax
from jax.experimental import pallas as pl
from jax.experimental.pallas import tpu as pltpu
from jax.experimental.pallas import tpu_sc as plsc

NC = 2
NS = 16
L = 16
NW = NC * NS

N = 100000
D = 128

TC_BLOCK = 8192
TC_GRID = (N + TC_BLOCK - 1) // TC_BLOCK
SIM_ROWS = TC_GRID * TC_BLOCK // 128
WROWS = SIM_ROWS // NW

_NEG_INF = float("-inf")
_IMAX = jnp.iinfo(jnp.int32).max


def _tc_sims(q_ref, keys_ref, out_ref):
    q = q_ref[0, :]
    qn = jnp.maximum(jnp.sqrt(jnp.sum(q * q)), 1e-8)
    k = keys_ref[...]
    q2 = q_ref[...].reshape(D, 1)
    dots = jnp.dot(k, q2, preferred_element_type=jnp.float32)
    ssq = jnp.dot(k * k, jnp.ones((D, 1), jnp.float32),
                  preferred_element_type=jnp.float32)
    kn = jnp.maximum(jnp.sqrt(ssq), 1e-8)
    sim = dots / (qn * kn)
    out_ref[...] = sim.reshape(TC_BLOCK // 128, 128)


def _sc_body(sims_hbm, values_hbm,
             rows_out, vals_out, idx_out,
             svmem, cand_v, cand_i, merged_v, merged_i, row_vmem,
             shared_v, shared_i):
    cid = lax.axis_index("c")
    sid = lax.axis_index("s")
    w = cid * NS + sid

    pltpu.sync_copy(sims_hbm.at[pl.ds(w * WROWS, WROWS)], svmem)

    wbase = w * (WROWS * 128) + lax.iota(jnp.int32, L)
    bv = jnp.full((L,), _NEG_INF, jnp.float32)
    bi = jnp.zeros((L,), jnp.int32)
    for r in range(WROWS):
        for c in range(128 // L):
            v = svmem[r, pl.ds(c * L, L)]
            gidx = wbase + (r * 128 + c * L)
            upd = (v > bv) & (gidx < N)
            bv = jnp.where(upd, v, bv)
            bi = jnp.where(upd, gidx, bi)

    cand_v[...] = bv
    cand_i[...] = bi
    pltpu.sync_copy(cand_v, shared_v.at[sid])
    pltpu.sync_copy(cand_i, shared_i.at[sid])
    plsc.subcore_barrier()

    @pl.when(sid == 0)
    def _():
        pltpu.sync_copy(shared_v, merged_v)
        pltpu.sync_copy(shared_i, merged_i)
        bv = merged_v[0, :]
        bi = merged_i[0, :]
        for t in range(1, NS):
            v = merged_v[t, :]
            i = merged_i[t, :]
            upd = (v > bv) | ((v == bv) & (i < bi))
            bv = jnp.where(upd, v, bv)
            bi = jnp.where(upd, i, bi)
        m = jnp.max(bv)
        midx = jnp.min(jnp.where(bv == m, bi, _IMAX))
        pltpu.sync_copy(values_hbm.at[pl.ds(midx, 1)], row_vmem)
        pltpu.sync_copy(row_vmem, rows_out.at[pl.ds(cid, 1)])
        cand_v[...] = jnp.full((L,), m, jnp.float32)
        cand_i[...] = jnp.full((L,), midx, jnp.int32)
        pltpu.sync_copy(cand_v, vals_out.at[cid])
        pltpu.sync_copy(cand_i, idx_out.at[cid])


@jax.jit
def kernel(query, keys, values):
    sims = pl.pallas_call(
        _tc_sims,
        grid=(TC_GRID,),
        in_specs=[
            pl.BlockSpec((1, D), lambda i: (0, 0)),
            pl.BlockSpec((TC_BLOCK, D), lambda i: (i, 0)),
        ],
        out_specs=pl.BlockSpec((TC_BLOCK // 128, 128), lambda i: (i, 0)),
        out_shape=jax.ShapeDtypeStruct((SIM_ROWS, 128), jnp.float32),
    )(query.reshape(1, D), keys)

    mesh = plsc.VectorSubcoreMesh(core_axis_name="c", subcore_axis_name="s")
    rows, vals, idxs = pl.kernel(
        _sc_body,
        out_type=(
            jax.ShapeDtypeStruct((NC, D), jnp.float32),
            jax.ShapeDtypeStruct((NC, L), jnp.float32),
            jax.ShapeDtypeStruct((NC, L), jnp.int32),
        ),
        mesh=mesh,
        compiler_params=pltpu.CompilerParams(
            use_tc_tiling_on_sc=False, needs_layout_passes=False),
        scratch_types=[
            pltpu.VMEM((WROWS, 128), jnp.float32),
            pltpu.VMEM((L,), jnp.float32),
            pltpu.VMEM((L,), jnp.int32),
            pltpu.VMEM((NS, L), jnp.float32),
            pltpu.VMEM((NS, L), jnp.int32),
            pltpu.VMEM((1, D), jnp.float32),
            pltpu.VMEM_SHARED((NS, L), jnp.float32),
            pltpu.VMEM_SHARED((NS, L), jnp.int32),
        ],
    )(sims, values)

    v0, v1 = vals[0, 0], vals[1, 0]
    i0, i1 = idxs[0, 0], idxs[1, 0]
    pick0 = (v0 > v1) | ((v0 == v1) & (i0 <= i1))
    return jnp.where(pick0, rows[0], rows[1])
